# baseline (speedup 1.0000x reference)
import functools

import jax
import jax.numpy as jnp
from jax import lax
from jax.experimental import pallas as pl
from jax.experimental.pallas import tpu as pltpu
from jax.experimental.pallas import tpu_sc as plsc

NUM_CORES = 2
NUM_SUBCORES = 16
LANES = 16
NUM_WORKERS = NUM_CORES * NUM_SUBCORES

COL_GROUP = 128
BCP = 32
PACK = 4


def _sc_kernel(B, W, DEPTH, src_hbm, seq_hbm, pk_hbm, out_hbm,
               table_v, src_bufs, pk_bufs, out_bufs,
               tab_sem, in_sems, out_sems):
    n_col_groups = W // COL_GROUP
    n_row_groups = NUM_WORKERS // n_col_groups
    rows_per_worker = B // n_row_groups
    bc = BCP * PACK

    wid = lax.axis_index("s") * NUM_CORES + lax.axis_index("c")
    cw = lax.rem(wid, n_col_groups)
    rw = lax.div(wid, n_col_groups)
    c0 = cw * COL_GROUP
    r_base = rw * rows_per_worker
    pk_base = rw * (rows_per_worker // PACK)

    copies = []
    for d in range(DEPTH):
        copies.append(pltpu.async_copy(
            seq_hbm.at[d, pl.ds(c0, COL_GROUP)],
            table_v.at[pl.ds(d * COL_GROUP, COL_GROUP)], tab_sem))

    n_chunks = rows_per_worker // bc
    vecs_per_row = COL_GROUP // LANES
    col_offsets = [
        jnp.arange(LANES, dtype=jnp.int32) + j * LANES
        for j in range(vecs_per_row)
    ]

    def src_rows(g):
        return pl.ds(r_base + g * bc, bc)

    def pk_rows(g):
        return pl.ds(pl.multiple_of(pk_base + g * BCP, BCP), BCP)

    def start_in(g, b):
        pltpu.async_copy(src_hbm.at[src_rows(g), pl.ds(c0, COL_GROUP)],
                         src_bufs[b], in_sems[b])
        pltpu.async_copy(pk_hbm.at[pk_rows(g), pl.ds(c0, COL_GROUP)],
                         pk_bufs[b], in_sems[b])

    def wait_in(g, b):
        pltpu.make_async_copy(src_hbm.at[src_rows(g), pl.ds(c0, COL_GROUP)],
                              src_bufs[b], in_sems[b]).wait()
        pltpu.make_async_copy(pk_hbm.at[pk_rows(g), pl.ds(c0, COL_GROUP)],
                              pk_bufs[b], in_sems[b]).wait()

    start_in(0, 0)
    start_in(1, 1)
    for cp in copies:
        cp.wait()

    def process(g, b):
        wait_in(g, b)

        @pl.when(g >= 2)
        def _():
            pltpu.make_async_copy(
                out_bufs[b], out_hbm.at[src_rows(g), pl.ds(c0, COL_GROUP)],
                out_sems[b]).wait()

        src_v, pk_v, out_v = src_bufs[b], pk_bufs[b], out_bufs[b]

        @plsc.parallel_loop(0, BCP, unroll=2)
        def _(pr):
            for j in range(vecs_per_row):
                sl = pl.ds(j * LANES, LANES)
                w = pk_v[pr, sl]
                for r in range(PACK):
                    iv = lax.shift_right_logical(w, 8 * r) & 0xFF
                    flat = iv * COL_GROUP + col_offsets[j]
                    gv = plsc.load_gather(table_v, [flat])
                    sv = src_v[pr * PACK + r, sl]
                    out_v[pr * PACK + r, sl] = jnp.where(
                        sv > gv, 1.0, 0.0).astype(jnp.float32)

        pltpu.async_copy(out_v, out_hbm.at[src_rows(g), pl.ds(c0, COL_GROUP)],
                         out_sems[b])

        @pl.when(g + 2 < n_chunks)
        def _():
            start_in(g + 2, b)

    def pair_body(p, _):
        process(p * 2, 0)
        process(p * 2 + 1, 1)
        return 0

    lax.fori_loop(0, n_chunks // 2, pair_body, 0)

    for b in range(2):
        g = n_chunks - 2 + b
        pltpu.make_async_copy(
            out_bufs[b], out_hbm.at[src_rows(g), pl.ds(c0, COL_GROUP)],
            out_sems[b]).wait()


def kernel(source, rng_seq, rng_idx):
    B, W = source.shape
    DEPTH = rng_seq.shape[0]

    idx = rng_idx.astype(jnp.int32)
    packed = (idx[0::4] | (idx[1::4] << 8) | (idx[2::4] << 16)
              | (idx[3::4] << 24))

    mesh = plsc.VectorSubcoreMesh(
        core_axis_name="c", subcore_axis_name="s",
        num_cores=NUM_CORES, num_subcores=NUM_SUBCORES)
    bc = BCP * PACK
    f = pl.kernel(
        functools.partial(_sc_kernel, B, W, DEPTH),
        out_type=jax.ShapeDtypeStruct((B, W), jnp.float32),
        mesh=mesh,
        scratch_types=[
            pltpu.VMEM((DEPTH * COL_GROUP,), jnp.float32),
            [pltpu.VMEM((bc, COL_GROUP), jnp.float32)] * 2,
            [pltpu.VMEM((BCP, COL_GROUP), jnp.int32)] * 2,
            [pltpu.VMEM((bc, COL_GROUP), jnp.float32)] * 2,
            pltpu.SemaphoreType.DMA,
            [pltpu.SemaphoreType.DMA] * 2,
            [pltpu.SemaphoreType.DMA] * 2,
        ],
        compiler_params=pltpu.CompilerParams(needs_layout_passes=False),
    )
    return f(source, rng_seq, packed)


# speedup vs baseline: 4.6416x; 4.6416x over previous
import functools

import jax
import jax.numpy as jnp
from jax import lax
from jax.experimental import pallas as pl
from jax.experimental.pallas import tpu as pltpu
from jax.experimental.pallas import tpu_sc as plsc

NUM_CORES = 2
NUM_SUBCORES = 16
LANES = 16
NUM_WORKERS = NUM_CORES * NUM_SUBCORES

COL_GROUP = 128
BC = 64
NBUF = 3


def _sc_kernel(B, W, DEPTH, src_hbm, seq_hbm, idx_hbm, out_hbm,
               table_v, src_bufs, idx_bufs, out_bufs,
               tab_sem, in_sems, out_sems):
    n_col_groups = W // COL_GROUP
    n_row_groups = NUM_WORKERS // n_col_groups
    rows_per_worker = B // n_row_groups

    wid = lax.axis_index("s") * NUM_CORES + lax.axis_index("c")
    cw = lax.rem(wid, n_col_groups)
    rw = lax.div(wid, n_col_groups)
    c0 = cw * COL_GROUP
    r_base = rw * rows_per_worker

    copies = []
    for d in range(DEPTH):
        copies.append(pltpu.async_copy(
            seq_hbm.at[d, pl.ds(c0, COL_GROUP)],
            table_v.at[pl.ds(d * COL_GROUP, COL_GROUP)], tab_sem))

    n_chunks = rows_per_worker // BC
    vecs_per_row = COL_GROUP // LANES
    col_offsets = [
        jnp.arange(LANES, dtype=jnp.int32) + j * LANES
        for j in range(vecs_per_row)
    ]

    def rows_of(g):
        return pl.ds(r_base + g * BC, BC)

    def start_in(g, b):
        pltpu.async_copy(src_hbm.at[rows_of(g), pl.ds(c0, COL_GROUP)],
                         src_bufs[b], in_sems[b])
        pltpu.async_copy(idx_hbm.at[rows_of(g), pl.ds(c0, COL_GROUP)],
                         idx_bufs[b], in_sems[b])

    def wait_in(g, b):
        pltpu.make_async_copy(src_hbm.at[rows_of(g), pl.ds(c0, COL_GROUP)],
                              src_bufs[b], in_sems[b]).wait()
        pltpu.make_async_copy(idx_hbm.at[rows_of(g), pl.ds(c0, COL_GROUP)],
                              idx_bufs[b], in_sems[b]).wait()

    for b in range(NBUF):
        start_in(b, b)
    for cp in copies:
        cp.wait()

    def process(g, b):
        wait_in(g, b)

        @pl.when(g >= NBUF)
        def _():
            pltpu.make_async_copy(
                out_bufs[b], out_hbm.at[rows_of(g), pl.ds(c0, COL_GROUP)],
                out_sems[b]).wait()

        src_v, idx_v, out_v = src_bufs[b], idx_bufs[b], out_bufs[b]

        @plsc.parallel_loop(0, BC, unroll=4)
        def _(i):
            for j in range(vecs_per_row):
                sl = pl.ds(j * LANES, LANES)
                iv = idx_v[i, sl]
                flat = iv * COL_GROUP + col_offsets[j]
                gv = plsc.load_gather(table_v, [flat])
                sv = src_v[i, sl]
                out_v[i, sl] = jnp.where(sv > gv, 1.0, 0.0).astype(jnp.float32)

        pltpu.async_copy(out_v, out_hbm.at[rows_of(g), pl.ds(c0, COL_GROUP)],
                         out_sems[b])

        @pl.when(g + NBUF < n_chunks)
        def _():
            start_in(g + NBUF, b)

    n_full = (n_chunks // NBUF) * NBUF

    def ring_body(p, _):
        for b in range(NBUF):
            process(p * NBUF + b, b)
        return 0

    lax.fori_loop(0, n_chunks // NBUF, ring_body, 0)
    for g in range(n_full, n_chunks):
        process(g, g % NBUF)

    for g in range(n_chunks - NBUF, n_chunks):
        pltpu.make_async_copy(
            out_bufs[g % NBUF], out_hbm.at[rows_of(g), pl.ds(c0, COL_GROUP)],
            out_sems[g % NBUF]).wait()


def kernel(source, rng_seq, rng_idx):
    B, W = source.shape
    DEPTH = rng_seq.shape[0]

    mesh = plsc.VectorSubcoreMesh(
        core_axis_name="c", subcore_axis_name="s",
        num_cores=NUM_CORES, num_subcores=NUM_SUBCORES)
    f = pl.kernel(
        functools.partial(_sc_kernel, B, W, DEPTH),
        out_type=jax.ShapeDtypeStruct((B, W), jnp.float32),
        mesh=mesh,
        scratch_types=[
            pltpu.VMEM((DEPTH * COL_GROUP,), jnp.float32),
            [pltpu.VMEM((BC, COL_GROUP), jnp.float32)] * NBUF,
            [pltpu.VMEM((BC, COL_GROUP), jnp.int32)] * NBUF,
            [pltpu.VMEM((BC, COL_GROUP), jnp.float32)] * NBUF,
            pltpu.SemaphoreType.DMA,
            [pltpu.SemaphoreType.DMA] * NBUF,
            [pltpu.SemaphoreType.DMA] * NBUF,
        ],
        compiler_params=pltpu.CompilerParams(needs_layout_passes=False),
    )
    return f(source, rng_seq, rng_idx)
